# baseline (speedup 1.0000x reference)
import functools

import jax
import jax.numpy as jnp
from jax import lax
from jax.experimental import pallas as pl
from jax.experimental.pallas import tpu as pltpu
from jax.experimental.pallas import tpu_sc as plsc

B, N, D = 1024, 128, 128
BC = 8
SPAN = 16

NC, NS = 2, 16
NW = NC * NS
BPW = B // NW
ZWORDS = N * D
L = 16


def _col_body(rand_ref, col_ref):
    r = rand_ref[...]
    rj = r[:, :, None, :]
    rk = r[:, :, :, None]
    lt = rk < rj
    ranks = jnp.sum(lt.astype(jnp.int32), axis=2)
    n_iota = lax.broadcasted_iota(jnp.int32, (SPAN, BC, N, N), 2)
    col_ref[...] = (ranks[:, :, None, :] == n_iota).astype(jnp.float32)


def _sc_row_body(out_hbm, zbuf, sem):
    wid = lax.axis_index("s") * NC + lax.axis_index("c")
    base = wid * BPW * ZWORDS

    def zstep(i, carry):
        for u in range(8):
            zbuf[pl.ds(i * (8 * L) + u * L, L)] = jnp.zeros((L,), jnp.float32)
        return carry

    lax.fori_loop(0, ZWORDS // (8 * L), zstep, 0)
    copies = [
        pltpu.async_copy(zbuf, out_hbm.at[pl.ds(base + t * ZWORDS, ZWORDS)], sem)
        for t in range(BPW)
    ]
    for cp in copies:
        cp.wait()


_sc_row = functools.partial(
    pl.kernel,
    out_type=jax.ShapeDtypeStruct((B * N * D,), jnp.float32),
    mesh=plsc.VectorSubcoreMesh(core_axis_name="c", subcore_axis_name="s"),
    scratch_types=[
        pltpu.VMEM((ZWORDS,), jnp.float32),
        pltpu.SemaphoreType.DMA,
    ],
)(_sc_row_body)


def kernel(distance_matrix):
    rand = jax.random.uniform(jax.random.key(42), (B, N), dtype=jnp.float32)
    row_flat = _sc_row()
    part = B // SPAN
    col = pl.pallas_call(
        _col_body,
        grid=(part // BC,),
        in_specs=[pl.BlockSpec((SPAN, BC, N), lambda i: (0, i, 0))],
        out_specs=pl.BlockSpec((SPAN, BC, N, D), lambda i: (0, i, 0, 0)),
        out_shape=jax.ShapeDtypeStruct((SPAN, part, N, D), jnp.float32),
    )(rand.reshape(SPAN, part, N))
    return (row_flat.reshape(B, N, D), col.reshape(B, N, D), distance_matrix)


# speedup vs baseline: 1.0066x; 1.0066x over previous
import functools

import jax
import jax.numpy as jnp
from jax import lax
from jax.experimental import pallas as pl
from jax.experimental.pallas import tpu as pltpu
from jax.experimental.pallas import tpu_sc as plsc

B, N, D = 1024, 128, 128
BC = 32
SPAN = 8

NC, NS = 2, 16
NW = NC * NS
BPW = B // NW
ZWORDS = N * D
L = 16


def _col_body(rand_ref, col_ref):
    r = rand_ref[...]
    rj = r[:, :, None, :]
    rk = r[:, :, :, None]
    lt = rk < rj
    ranks = jnp.sum(lt.astype(jnp.int32), axis=2)
    n_iota = lax.broadcasted_iota(jnp.int32, (SPAN, BC, N, N), 2)
    col_ref[...] = (ranks[:, :, None, :] == n_iota).astype(jnp.float32)


def _sc_row_body(out_hbm, zbuf, sem):
    wid = lax.axis_index("s") * NC + lax.axis_index("c")
    base = wid * BPW * ZWORDS

    def zstep(i, carry):
        for u in range(8):
            zbuf[pl.ds(i * (8 * L) + u * L, L)] = jnp.zeros((L,), jnp.float32)
        return carry

    lax.fori_loop(0, ZWORDS // (8 * L), zstep, 0)
    copies = [
        pltpu.async_copy(zbuf, out_hbm.at[pl.ds(base + t * ZWORDS, ZWORDS)], sem)
        for t in range(BPW)
    ]
    for cp in copies:
        cp.wait()


_sc_row = functools.partial(
    pl.kernel,
    out_type=jax.ShapeDtypeStruct((B * N * D,), jnp.float32),
    mesh=plsc.VectorSubcoreMesh(core_axis_name="c", subcore_axis_name="s"),
    scratch_types=[
        pltpu.VMEM((ZWORDS,), jnp.float32),
        pltpu.SemaphoreType.DMA,
    ],
)(_sc_row_body)


def kernel(distance_matrix):
    rand = jax.random.uniform(jax.random.key(42), (B, N), dtype=jnp.float32)
    row_flat = _sc_row()
    part = B // SPAN
    col = pl.pallas_call(
        _col_body,
        grid=(part // BC,),
        in_specs=[pl.BlockSpec((SPAN, BC, N), lambda i: (0, i, 0))],
        out_specs=pl.BlockSpec((SPAN, BC, N, D), lambda i: (0, i, 0, 0)),
        out_shape=jax.ShapeDtypeStruct((SPAN, part, N, D), jnp.float32),
    )(rand.reshape(SPAN, part, N))
    return (row_flat.reshape(B, N, D), col.reshape(B, N, D), distance_matrix)
